# baseline (speedup 1.0000x reference)
import functools

import numpy as np
import jax
import jax.numpy as jnp
from jax import lax
from jax.experimental import pallas as pl
from jax.experimental.pallas import tpu as pltpu
from jax.experimental.pallas import tpu_sc as plsc

_D = 64
_SEQ = 200
_BATCH = 4096

_NC = 2
_NS = 16
_NW = _NC * _NS
_LANES = 128
_SGRP = _SEQ // 8


def _pos_encoding(length, depth):
    d = depth // 2
    positions = np.arange(length)[:, np.newaxis]
    depths = np.arange(d)[np.newaxis, :] / d
    rads = positions / 10000 ** depths
    pe = np.concatenate([np.sin(rads), np.cos(rads)], axis=-1)
    return jnp.asarray(pe, dtype=jnp.float32)


def _sc_embed(table, xt, pe):
    mesh = plsc.VectorSubcoreMesh(core_axis_name="c", subcore_axis_name="s")

    @functools.partial(
        pl.kernel,
        mesh=mesh,
        compiler_params=pltpu.CompilerParams(
            use_tc_tiling_on_sc=False, needs_layout_passes=False),
        out_type=jax.ShapeDtypeStruct((_SEQ, 8, _NW, 1024), jnp.float32),
        scratch_types=[
            pltpu.VMEM((_SGRP, 8, _LANES), jnp.int32),
            pltpu.VMEM((_SEQ, _D), jnp.float32),
            pltpu.VMEM((2, _LANES, _D), jnp.float32),
            pltpu.VMEM((8192,), jnp.float32),
            pltpu.VMEM((8192,), jnp.float32),
            pltpu.SemaphoreType.DMA,
            pltpu.SemaphoreType.DMA,
            pltpu.SemaphoreType.DMA,
            pltpu.SemaphoreType.DMA,
        ],
    )
    def k(table_hbm, xt_hbm, pe_hbm, out_hbm, idx_v, pe_v, gbuf, tbuf0,
          tbuf1, gsem0, gsem1, osem0, osem1):
        tbufs = (tbuf0, tbuf1)
        wid = lax.axis_index("s") * _NC + lax.axis_index("c")
        pltpu.sync_copy(xt_hbm.at[:, wid], idx_v)
        pltpu.sync_copy(pe_hbm, pe_v)

        iota = lax.iota(jnp.int32, 16)
        flatbase_vecs = []
        for dg in range(4):
            d_vec = dg * 16 + iota
            flatbase_vecs.append(((d_vec >> 3) << 10) + ((d_vec & 7) << 7))

        def gather_copy(s, b, sem):
            return pltpu.make_async_copy(
                table_hbm.at[idx_v.at[s // 8, s % 8]], gbuf.at[b], sem)

        def out_copy(s, b, r, sem):
            return pltpu.make_async_copy(
                tbufs[b].at[pl.ds(r * 1024, 1024)], out_hbm.at[s, r, wid], sem)

        gather_copy(0, 0, gsem0).start()

        def body(g, carry):
            for b in range(2):
                s = 2 * g + b
                gsem = gsem0 if b == 0 else gsem1
                osem = osem0 if b == 0 else osem1
                ogsem = gsem1 if b == 0 else gsem0

                @pl.when(s < _SEQ - 1)
                def _():
                    gather_copy(s + 1, 1 - b, ogsem).start()

                gather_copy(s, b, gsem).wait()

                @pl.when(s >= 2)
                def _():
                    for r in range(8):
                        out_copy(s - 2, b, r, osem).wait()

                pe_vregs = [pe_v[s, pl.ds(dg * 16, 16)] for dg in range(4)]
                tbuf2d = tbufs[b]

                def ebody(e, c2):
                    esplat = jnp.full((16,), e, jnp.int32)
                    for dg in range(4):
                        v = gbuf[b, e, pl.ds(dg * 16, 16)] + pe_vregs[dg]
                        plsc.store_scatter(
                            tbuf2d, [flatbase_vecs[dg] + esplat], v)
                    return c2

                lax.fori_loop(0, _LANES, ebody, 0, unroll=2)
                for r in range(8):
                    out_copy(s, b, r, osem).start()
            return carry

        lax.fori_loop(0, _SEQ // 2, body, 0)
        for r in range(8):
            out_copy(_SEQ - 2, 0, r, osem0).wait()
        for r in range(8):
            out_copy(_SEQ - 1, 1, r, osem1).wait()

    return k(table, xt, pe)


def kernel(x, table):
    xt = (x.astype(jnp.int32).T
          .reshape(_SGRP, 8, _NW, _LANES).transpose(0, 2, 1, 3))
    pe = _pos_encoding(_SEQ, _D)
    out4d = _sc_embed(table, xt, pe)
    out = (out4d.reshape(_SEQ, 8, _NW, 8, _LANES)
           .transpose(2, 4, 0, 1, 3).reshape(_BATCH, _SEQ, _D))
    return out


# speedup vs baseline: 1.3208x; 1.3208x over previous
import functools

import numpy as np
import jax
import jax.numpy as jnp
from jax import lax
from jax.experimental import pallas as pl
from jax.experimental.pallas import tpu as pltpu
from jax.experimental.pallas import tpu_sc as plsc

_D = 64
_SEQ = 200
_BATCH = 4096

_NC = 2
_NS = 16
_NW = _NC * _NS
_LANES = 128
_SGRP = _SEQ // 8


def _pos_encoding(length, depth):
    d = depth // 2
    positions = np.arange(length)[:, np.newaxis]
    depths = np.arange(d)[np.newaxis, :] / d
    rads = positions / 10000 ** depths
    pe = np.concatenate([np.sin(rads), np.cos(rads)], axis=-1)
    return jnp.asarray(pe, dtype=jnp.float32)


def _sc_embed(table, xt, pe):
    mesh = plsc.VectorSubcoreMesh(core_axis_name="c", subcore_axis_name="s")

    @functools.partial(
        pl.kernel,
        mesh=mesh,
        compiler_params=pltpu.CompilerParams(
            use_tc_tiling_on_sc=False, needs_layout_passes=False),
        out_type=jax.ShapeDtypeStruct((_SEQ, 8, _NW, 1024), jnp.float32),
        scratch_types=[
            pltpu.VMEM((_SGRP, 8, _LANES), jnp.int32),
            pltpu.VMEM((_SEQ, _D), jnp.float32),
            pltpu.VMEM((2, _LANES, _D), jnp.float32),
            pltpu.VMEM((8192,), jnp.float32),
            pltpu.VMEM((8192,), jnp.float32),
            pltpu.SemaphoreType.DMA,
            pltpu.SemaphoreType.DMA,
            pltpu.SemaphoreType.DMA,
            pltpu.SemaphoreType.DMA,
        ],
    )
    def k(table_hbm, xt_hbm, pe_hbm, out_hbm, idx_v, pe_v, gbuf, tbuf0,
          tbuf1, gsem0, gsem1, osem0, osem1):
        tbufs = (tbuf0, tbuf1)
        wid = lax.axis_index("s") * _NC + lax.axis_index("c")
        pltpu.sync_copy(xt_hbm.at[:, wid], idx_v)
        pltpu.sync_copy(pe_hbm, pe_v)

        iota = lax.iota(jnp.int32, 16)
        flatbase_vecs = []
        for dg in range(4):
            d_vec = dg * 16 + iota
            flatbase_vecs.append(((d_vec >> 3) << 10) + ((d_vec & 7) << 7))

        def gather_copy(s, b, sem):
            return pltpu.make_async_copy(
                table_hbm.at[idx_v.at[s // 8, s % 8]], gbuf.at[b], sem)

        def out_copy(s, b, r, sem):
            return pltpu.make_async_copy(
                tbufs[b].at[pl.ds(r * 1024, 1024)], out_hbm.at[s, r, wid], sem)

        gather_copy(0, 0, gsem0).start()

        def body(g, carry):
            for b in range(2):
                s = 2 * g + b
                gsem = gsem0 if b == 0 else gsem1
                osem = osem0 if b == 0 else osem1
                ogsem = gsem1 if b == 0 else gsem0

                @pl.when(s < _SEQ - 1)
                def _():
                    gather_copy(s + 1, 1 - b, ogsem).start()

                gather_copy(s, b, gsem).wait()

                @pl.when(s >= 2)
                def _():
                    for r in range(8):
                        out_copy(s - 2, b, r, osem).wait()

                pe_vregs = [pe_v[s, pl.ds(dg * 16, 16)] for dg in range(4)]
                tbuf2d = tbufs[b]

                @plsc.parallel_loop(0, _LANES, unroll=4)
                def _(e):
                    esplat = jnp.full((16,), e, jnp.int32)
                    for dg in range(4):
                        v = gbuf[b, e, pl.ds(dg * 16, 16)] + pe_vregs[dg]
                        plsc.store_scatter(
                            tbuf2d, [flatbase_vecs[dg] + esplat], v)
                for r in range(8):
                    out_copy(s, b, r, osem).start()
            return carry

        lax.fori_loop(0, _SEQ // 2, body, 0)
        for r in range(8):
            out_copy(_SEQ - 2, 0, r, osem0).wait()
        for r in range(8):
            out_copy(_SEQ - 1, 1, r, osem1).wait()

    return k(table, xt, pe)


def kernel(x, table):
    xt = (x.astype(jnp.int32).T
          .reshape(_SGRP, 8, _NW, _LANES).transpose(0, 2, 1, 3))
    pe = _pos_encoding(_SEQ, _D)
    out4d = _sc_embed(table, xt, pe)
    out = (out4d.reshape(_SEQ, 8, _NW, 8, _LANES)
           .transpose(2, 4, 0, 1, 3).reshape(_BATCH, _SEQ, _D))
    return out
